# initial kernel scaffold (unmeasured)
import jax
import jax.numpy as jnp
from jax import lax
from jax.experimental import pallas as pl
from jax.experimental.pallas import tpu as pltpu


def kernel(
    x,
):
    def body(*refs):
        pass

    out_shape = jax.ShapeDtypeStruct(..., jnp.float32)
    return pl.pallas_call(body, out_shape=out_shape)(...)


# baseline (device time: 79719 ns/iter reference)
import jax
import jax.numpy as jnp
from jax import lax
from jax.experimental import pallas as pl
from jax.experimental.pallas import tpu as pltpu

N_Z = 4


def kernel(x):
    m, n = x.shape
    n_out = n // N_Z

    def body(x_ref, out_ref, send_sems, recv_sems):
        my_x = lax.axis_index("x")
        my_y = lax.axis_index("y")
        my_z = lax.axis_index("z")

        barrier_sem = pltpu.get_barrier_semaphore()
        for d in range(1, N_Z):
            peer = (my_z + d) % N_Z
            pl.semaphore_signal(
                barrier_sem, inc=1,
                device_id=(my_x, my_y, peer),
                device_id_type=pl.DeviceIdType.MESH,
            )
        pl.semaphore_wait(barrier_sem, N_Z - 1)

        out_ref[pl.ds(my_z * m, m), :] = x_ref[:, pl.ds(my_z * n_out, n_out)]

        rdmas = []
        for d in range(1, N_Z):
            peer = (my_z + d) % N_Z
            rdma = pltpu.make_async_remote_copy(
                src_ref=x_ref.at[:, pl.ds(peer * n_out, n_out)],
                dst_ref=out_ref.at[pl.ds(my_z * m, m), :],
                send_sem=send_sems.at[d - 1],
                recv_sem=recv_sems.at[d - 1],
                device_id=(my_x, my_y, peer),
                device_id_type=pl.DeviceIdType.MESH,
            )
            rdma.start()
            rdmas.append(rdma)

        for rdma in rdmas:
            rdma.wait()

    return pl.pallas_call(
        body,
        out_shape=jax.ShapeDtypeStruct((N_Z * m, n_out), x.dtype),
        in_specs=[pl.BlockSpec(memory_space=pltpu.VMEM)],
        out_specs=pl.BlockSpec(memory_space=pltpu.VMEM),
        scratch_shapes=[
            pltpu.SemaphoreType.DMA((N_Z - 1,)),
            pltpu.SemaphoreType.DMA((N_Z - 1,)),
        ],
        compiler_params=pltpu.CompilerParams(collective_id=0),
    )(x)
